# baseline (speedup 1.0000x reference)
import functools

import jax
import jax.numpy as jnp
import numpy as np
from jax import lax
from jax.experimental import pallas as pl
from jax.experimental.pallas import tpu as pltpu
from jax.experimental.pallas import tpu_sc as plsc

N = 10000
E = 320000
C = 128
H = C // 2
T = 2
NS = 16
NC = 2
B = 128
EPT = 20480
NB = EPT // B
E_PAD = NS * EPT
ROWS = 20096
RPT = ROWS // NS


NPH = 1
PB = NB // NPH
CB = 2 * B


def _sc_segment_sum(xr, gix_r, six_r):
    mesh = plsc.VectorSubcoreMesh(core_axis_name="c", subcore_axis_name="s",
                                  num_cores=NC, num_subcores=NS)

    @functools.partial(
        pl.kernel,
        mesh=mesh,
        out_type=jax.ShapeDtypeStruct((NC, ROWS, H), jnp.float32),
        scratch_types=[
            pltpu.VMEM((PB, B), jnp.int32),
            pltpu.VMEM((PB, B), jnp.int32),
            pltpu.VMEM((B, H), jnp.float32),
            pltpu.VMEM_SHARED((ROWS, H), jnp.float32),
            pltpu.SemaphoreType.DMA,
        ],
        compiler_params=pltpu.CompilerParams(use_tc_tiling_on_sc=False),
    )
    def body(xr_h, gix_h, six_h, out_h, gix_v, six_v, f2,
             agg_s, sem_g):
        c = lax.axis_index("c")
        s = lax.axis_index("s")

        z16 = jnp.zeros((16,), jnp.float32)

        def zvbody(i, _):
            for l in range(H // 16):
                f2[i, pl.ds(l * 16, 16)] = z16
            return 0

        lax.fori_loop(0, B, zvbody, 0)

        zbase = s * RPT

        def zdbody(k, _):
            start = zbase + jnp.minimum(k * B, RPT - B)
            pltpu.sync_copy(f2, agg_s.at[pl.ds(start, B)])
            return 0

        lax.fori_loop(0, (RPT + B - 1) // B, zdbody, 0)

        plsc.subcore_barrier()

        pltpu.sync_copy(gix_h.at[c, s], gix_v)
        pltpu.sync_copy(six_h.at[s], six_v)

        def mainbody(j, _):
            pltpu.async_copy(xr_h.at[gix_v.at[j]], f2, sem_g).wait()
            pltpu.sync_copy(f2, agg_s.at[six_v.at[j]], add=True)
            return 0

        lax.fori_loop(0, NB, mainbody, 0)

        plsc.subcore_barrier()

        def obody(k, _):
            start = zbase + jnp.minimum(k * CB, RPT - CB)
            pltpu.sync_copy(agg_s.at[pl.ds(start, CB)],
                            out_h.at[c, pl.ds(start, CB)])
            return 0

        lax.fori_loop(0, (RPT + CB - 1) // CB, obody, 0)

    return body(xr, gix_r, six_r)


def _tc_dense(x, agg, weight, rw, bias):
    BLK = 1000
    nbk = N // BLK

    def body(x_b, a00, a01, a10, a11, w, rw_b, b_b, o):
        acc = jnp.dot(x_b[...], rw_b[...], preferred_element_type=jnp.float32)
        acc += jnp.dot(a00[0], w[0, :H, :], preferred_element_type=jnp.float32)
        acc += jnp.dot(a10[0], w[0, H:, :], preferred_element_type=jnp.float32)
        acc += jnp.dot(a01[0], w[1, :H, :], preferred_element_type=jnp.float32)
        acc += jnp.dot(a11[0], w[1, H:, :], preferred_element_type=jnp.float32)
        o[...] = acc + b_b[...]

    def agg_spec(t, h):
        return pl.BlockSpec((1, BLK, H),
                            lambda i, _t=t, _h=h: (_h, i + _t * nbk, 0))

    return pl.pallas_call(
        body,
        grid=(nbk,),
        in_specs=[
            pl.BlockSpec((BLK, C), lambda i: (i, 0)),
            agg_spec(0, 0),
            agg_spec(1, 0),
            agg_spec(0, 1),
            agg_spec(1, 1),
            pl.BlockSpec((T, C, C), lambda i: (0, 0, 0)),
            pl.BlockSpec((C, C), lambda i: (0, 0)),
            pl.BlockSpec((1, C), lambda i: (0, 0)),
        ],
        out_specs=pl.BlockSpec((BLK, C), lambda i: (i, 0)),
        out_shape=jax.ShapeDtypeStruct((N, C), jnp.float32),
    )(x, agg, agg, agg, agg, weight, rw, bias)


@jax.jit
def kernel(x, edge_index, edge_type, weight, root_w, root_b):
    src = edge_index[0]
    dst = edge_index[1]

    xr = x.reshape(T * N, H)
    pad = E_PAD - E
    src_p = jnp.concatenate([src, jnp.zeros((pad,), jnp.int32)])
    gix_r = jnp.stack([2 * src_p, 2 * src_p + 1]).reshape(NC, NS, NB, B)
    six = edge_type * N + dst
    six_r = jnp.concatenate([six, jnp.full((pad,), T * N, jnp.int32)]).reshape(
        NS, NB, B)

    agg = _sc_segment_sum(xr, gix_r, six_r)
    return _tc_dense(x, agg, weight, root_w.T, root_b.reshape(1, C))


# speedup vs baseline: 1.0670x; 1.0670x over previous
import functools

import jax
import jax.numpy as jnp
import numpy as np
from jax import lax
from jax.experimental import pallas as pl
from jax.experimental.pallas import tpu as pltpu
from jax.experimental.pallas import tpu_sc as plsc

N = 10000
E = 320000
C = 128
H = C // 2
T = 2
NS = 16
NC = 2
B = 128
EPT = 20480
NB = EPT // B
E_PAD = NS * EPT
ROWS = 20096
RPT = ROWS // NS


NPH = 1
PB = NB // NPH
CB = 2 * B


def _sc_segment_sum(xr, gix_r, six_r):
    mesh = plsc.VectorSubcoreMesh(core_axis_name="c", subcore_axis_name="s",
                                  num_cores=NC, num_subcores=NS)

    @functools.partial(
        pl.kernel,
        mesh=mesh,
        out_type=jax.ShapeDtypeStruct((NC, ROWS, H), jnp.float32),
        scratch_types=[
            pltpu.VMEM((PB, B), jnp.int32),
            pltpu.VMEM((PB, B), jnp.int32),
            pltpu.VMEM((B, H), jnp.float32),
            pltpu.VMEM_SHARED((ROWS, H), jnp.float32),
            pltpu.SemaphoreType.DMA,
        ],
        compiler_params=pltpu.CompilerParams(use_tc_tiling_on_sc=False),
    )
    def body(xr_h, gix_h, six_h, out_h, gix_v, six_v, f2,
             agg_s, sem_g):
        c = lax.axis_index("c")
        s = lax.axis_index("s")

        z16 = jnp.zeros((16,), jnp.float32)

        def zvbody(i, _):
            for l in range(H // 16):
                f2[i, pl.ds(l * 16, 16)] = z16
            return 0

        lax.fori_loop(0, B, zvbody, 0)

        zbase = s * RPT

        def zdbody(k, _):
            start = zbase + jnp.minimum(k * B, RPT - B)
            pltpu.sync_copy(f2, agg_s.at[pl.ds(start, B)])
            return 0

        lax.fori_loop(0, (RPT + B - 1) // B, zdbody, 0)

        plsc.subcore_barrier()

        pltpu.sync_copy(gix_h.at[c, s], gix_v)
        pltpu.sync_copy(six_h.at[s], six_v)

        def mainbody(j, _):
            pltpu.async_copy(xr_h.at[gix_v.at[j]], f2, sem_g).wait()
            pltpu.sync_copy(f2, agg_s.at[six_v.at[j]], add=True)
            return 0

        lax.fori_loop(0, NB, mainbody, 0)

        plsc.subcore_barrier()

        def obody(k, _):
            start = zbase + jnp.minimum(k * CB, RPT - CB)
            pltpu.sync_copy(agg_s.at[pl.ds(start, CB)],
                            out_h.at[c, pl.ds(start, CB)])
            return 0

        lax.fori_loop(0, (RPT + CB - 1) // CB, obody, 0)

    return body(xr, gix_r, six_r)


def _tc_dense(x, agg, weight, rw, bias):
    BLK = 1000
    nbk = N // BLK

    def body(x_b, a00, a01, a10, a11, w, rw_b, b_b, o):
        acc = jnp.dot(x_b[...], rw_b[...], preferred_element_type=jnp.float32)
        acc += jnp.dot(a00[0], w[0, :H, :], preferred_element_type=jnp.float32)
        acc += jnp.dot(a10[0], w[0, H:, :], preferred_element_type=jnp.float32)
        acc += jnp.dot(a01[0], w[1, :H, :], preferred_element_type=jnp.float32)
        acc += jnp.dot(a11[0], w[1, H:, :], preferred_element_type=jnp.float32)
        o[...] = acc + b_b[...]

    def agg_spec(t, h):
        return pl.BlockSpec((1, BLK, H),
                            lambda i, _t=t, _h=h: (_h, i + _t * nbk, 0))

    return pl.pallas_call(
        body,
        grid=(nbk,),
        in_specs=[
            pl.BlockSpec((BLK, C), lambda i: (i, 0)),
            agg_spec(0, 0),
            agg_spec(1, 0),
            agg_spec(0, 1),
            agg_spec(1, 1),
            pl.BlockSpec((T, C, C), lambda i: (0, 0, 0)),
            pl.BlockSpec((C, C), lambda i: (0, 0)),
            pl.BlockSpec((1, C), lambda i: (0, 0)),
        ],
        out_specs=pl.BlockSpec((BLK, C), lambda i: (i, 0)),
        out_shape=jax.ShapeDtypeStruct((N, C), jnp.float32),
    )(x, agg, agg, agg, agg, weight, rw, bias)


@jax.jit
def kernel(x, edge_index, edge_type, weight, root_w, root_b):
    src = edge_index[0]
    dst = edge_index[1]

    xr = jnp.concatenate([x[:, :H], x[:, H:]], axis=0)
    pad = E_PAD - E
    src_p = jnp.concatenate([src, jnp.zeros((pad,), jnp.int32)])
    gix_r = jnp.stack([src_p, src_p + N]).reshape(NC, NS, NB, B)
    six = edge_type * N + dst
    six_r = jnp.concatenate([six, jnp.full((pad,), T * N, jnp.int32)]).reshape(
        NS, NB, B)

    agg = _sc_segment_sum(xr, gix_r, six_r)
    return _tc_dense(x, agg, weight, root_w.T, root_b.reshape(1, C))


# speedup vs baseline: 1.5136x; 1.4186x over previous
import functools

import jax
import jax.numpy as jnp
import numpy as np
from jax import lax
from jax.experimental import pallas as pl
from jax.experimental.pallas import tpu as pltpu
from jax.experimental.pallas import tpu_sc as plsc

N = 10000
E = 320000
C = 128
H = C // 2
T = 2
NS = 16
NC = 2
B = 128
EPT = 20096
NB = EPT // B
E_PAD = NS * EPT
ROWS = 20096
RPT = ROWS // NS


NPH = 1
PB = NB // NPH
CB = 2 * B


def _sc_segment_sum(xr, gix_r, six_r):
    mesh = plsc.VectorSubcoreMesh(core_axis_name="c", subcore_axis_name="s",
                                  num_cores=NC, num_subcores=NS)

    @functools.partial(
        pl.kernel,
        mesh=mesh,
        out_type=jax.ShapeDtypeStruct((NC, ROWS, H), jnp.float32),
        scratch_types=[
            pltpu.VMEM((PB, B), jnp.int32),
            pltpu.VMEM((PB, B), jnp.int32),
            pltpu.VMEM((B, H), jnp.float32),
            pltpu.VMEM_SHARED((ROWS, H), jnp.float32),
            pltpu.SemaphoreType.DMA,
        ],
        compiler_params=pltpu.CompilerParams(use_tc_tiling_on_sc=False),
    )
    def body(xr_h, gix_h, six_h, out_h, gix_v, six_v, f2,
             agg_s, sem_g):
        c = lax.axis_index("c")
        s = lax.axis_index("s")

        z16 = jnp.zeros((16,), jnp.float32)

        def zvbody(i, _):
            for l in range(H // 16):
                f2[i, pl.ds(l * 16, 16)] = z16
            return 0

        lax.fori_loop(0, B, zvbody, 0)

        zbase = s * RPT

        def zdbody(k, _):
            start = zbase + jnp.minimum(k * B, RPT - B)
            pltpu.sync_copy(f2, agg_s.at[pl.ds(start, B)])
            return 0

        lax.fori_loop(0, (RPT + B - 1) // B, zdbody, 0)

        plsc.subcore_barrier()

        pltpu.sync_copy(gix_h.at[c, s], gix_v)
        pltpu.sync_copy(six_h.at[s], six_v)

        def mainbody(j, _):
            pltpu.async_copy(xr_h.at[gix_v.at[j]], f2, sem_g).wait()
            pltpu.sync_copy(f2, agg_s.at[six_v.at[j]], add=True)
            return 0

        lax.fori_loop(0, NB, mainbody, 0)

        plsc.subcore_barrier()

        def obody(k, _):
            start = zbase + jnp.minimum(k * CB, RPT - CB)
            pltpu.sync_copy(agg_s.at[pl.ds(start, CB)],
                            out_h.at[c, pl.ds(start, CB)])
            return 0

        lax.fori_loop(0, (RPT + CB - 1) // CB, obody, 0)

    return body(xr, gix_r, six_r)


def _tc_dense(x, agg, weight, rw, bias):
    BLK = 1000
    nbk = N // BLK

    def body(x_b, a00, a01, a10, a11, w, rw_b, b_b, o):
        acc = jnp.dot(x_b[...], rw_b[...], preferred_element_type=jnp.float32)
        acc += jnp.dot(a00[0], w[0, :H, :], preferred_element_type=jnp.float32)
        acc += jnp.dot(a10[0], w[0, H:, :], preferred_element_type=jnp.float32)
        acc += jnp.dot(a01[0], w[1, :H, :], preferred_element_type=jnp.float32)
        acc += jnp.dot(a11[0], w[1, H:, :], preferred_element_type=jnp.float32)
        o[...] = acc + b_b[...]

    def agg_spec(t, h):
        return pl.BlockSpec((1, BLK, H),
                            lambda i, _t=t, _h=h: (_h, i + _t * nbk, 0))

    return pl.pallas_call(
        body,
        grid=(nbk,),
        in_specs=[
            pl.BlockSpec((BLK, C), lambda i: (i, 0)),
            agg_spec(0, 0),
            agg_spec(1, 0),
            agg_spec(0, 1),
            agg_spec(1, 1),
            pl.BlockSpec((T, C, C), lambda i: (0, 0, 0)),
            pl.BlockSpec((C, C), lambda i: (0, 0)),
            pl.BlockSpec((1, C), lambda i: (0, 0)),
        ],
        out_specs=pl.BlockSpec((BLK, C), lambda i: (i, 0)),
        out_shape=jax.ShapeDtypeStruct((N, C), jnp.float32),
    )(x, agg, agg, agg, agg, weight, rw, bias)


@jax.jit
def kernel(x, edge_index, edge_type, weight, root_w, root_b):
    src = edge_index[0]
    dst = edge_index[1]

    xr = jnp.concatenate([x[:, :H], x[:, H:]], axis=0)
    pad = E_PAD - E
    src_p = jnp.concatenate([src, jnp.zeros((pad,), jnp.int32)])
    gix_r = jnp.stack([src_p, src_p + N]).reshape(NC, NS, NB, B)
    six = edge_type * N + dst
    trash = T * N + jnp.arange(pad, dtype=jnp.int32) % (ROWS - T * N)
    six_r = jnp.concatenate([six, trash]).reshape(NS, NB, B)

    agg = _sc_segment_sum(xr, gix_r, six_r)
    return _tc_dense(x, agg, weight, root_w.T, root_b.reshape(1, C))


# speedup vs baseline: 1.5875x; 1.0489x over previous
import functools

import jax
import jax.numpy as jnp
import numpy as np
from jax import lax
from jax.experimental import pallas as pl
from jax.experimental.pallas import tpu as pltpu
from jax.experimental.pallas import tpu_sc as plsc

N = 10000
E = 320000
C = 128
H = C // 2
T = 2
NS = 16
NC = 2
B = 128
EPT = 20480
NB = EPT // B
E_PAD = NS * EPT
ROWS = 20096
RPT = ROWS // NS
XROWS = 10240
XPT = XROWS // NS


NPH = 32
PB = NB // NPH
CB = 2 * B


def _sc_segment_sum(xr, gix_r, six_r):
    mesh = plsc.VectorSubcoreMesh(core_axis_name="c", subcore_axis_name="s",
                                  num_cores=NC, num_subcores=NS)

    @functools.partial(
        pl.kernel,
        mesh=mesh,
        out_type=jax.ShapeDtypeStruct((NC, ROWS, H), jnp.float32),
        scratch_types=[
            pltpu.VMEM((PB, B), jnp.int32),
            pltpu.VMEM((PB, B), jnp.int32),
            pltpu.VMEM((B, H), jnp.float32),
            pltpu.VMEM_SHARED((ROWS, H), jnp.float32),
            pltpu.VMEM_SHARED((XROWS, H), jnp.float32),
            pltpu.SemaphoreType.DMA,
        ],
        compiler_params=pltpu.CompilerParams(use_tc_tiling_on_sc=False),
    )
    def body(xr_h, gix_h, six_h, out_h, gix_v, six_v, f2,
             agg_s, xs_s, sem_g):
        c = lax.axis_index("c")
        s = lax.axis_index("s")

        def xbody(k, _):
            start = jnp.minimum(s * XPT + k * B, N - B)
            pltpu.sync_copy(xr_h.at[pl.ds(c * N + start, B)], f2)
            pltpu.sync_copy(f2, xs_s.at[pl.ds(start, B)])
            return 0

        lax.fori_loop(0, XPT // B, xbody, 0)

        z16 = jnp.zeros((16,), jnp.float32)

        def zvbody(i, _):
            for l in range(H // 16):
                f2[i, pl.ds(l * 16, 16)] = z16
            return 0

        lax.fori_loop(0, B, zvbody, 0)

        zbase = s * RPT

        def zdbody(k, _):
            start = zbase + jnp.minimum(k * B, RPT - B)
            pltpu.sync_copy(f2, agg_s.at[pl.ds(start, B)])
            return 0

        lax.fori_loop(0, (RPT + B - 1) // B, zdbody, 0)

        plsc.subcore_barrier()

        def mainbody(j, _):
            pltpu.async_copy(xs_s.at[gix_v.at[j]], f2, sem_g).wait()
            pltpu.sync_copy(f2, agg_s.at[six_v.at[j]], add=True)
            return 0

        for ph in range(NPH):
            pltpu.sync_copy(gix_h.at[s, pl.ds(ph * PB, PB)], gix_v)
            pltpu.sync_copy(six_h.at[s, pl.ds(ph * PB, PB)], six_v)
            lax.fori_loop(0, PB, mainbody, 0)

        plsc.subcore_barrier()

        def obody(k, _):
            start = zbase + jnp.minimum(k * CB, RPT - CB)
            pltpu.sync_copy(agg_s.at[pl.ds(start, CB)],
                            out_h.at[c, pl.ds(start, CB)])
            return 0

        lax.fori_loop(0, (RPT + CB - 1) // CB, obody, 0)

    return body(xr, gix_r, six_r)


def _tc_dense(x, agg, weight, rw, bias):
    BLK = 1000
    nbk = N // BLK

    def body(x_b, a00, a01, a10, a11, w, rw_b, b_b, o):
        acc = jnp.dot(x_b[...], rw_b[...], preferred_element_type=jnp.float32)
        acc += jnp.dot(a00[0], w[0, :H, :], preferred_element_type=jnp.float32)
        acc += jnp.dot(a10[0], w[0, H:, :], preferred_element_type=jnp.float32)
        acc += jnp.dot(a01[0], w[1, :H, :], preferred_element_type=jnp.float32)
        acc += jnp.dot(a11[0], w[1, H:, :], preferred_element_type=jnp.float32)
        o[...] = acc + b_b[...]

    def agg_spec(t, h):
        return pl.BlockSpec((1, BLK, H),
                            lambda i, _t=t, _h=h: (_h, i + _t * nbk, 0))

    return pl.pallas_call(
        body,
        grid=(nbk,),
        in_specs=[
            pl.BlockSpec((BLK, C), lambda i: (i, 0)),
            agg_spec(0, 0),
            agg_spec(1, 0),
            agg_spec(0, 1),
            agg_spec(1, 1),
            pl.BlockSpec((T, C, C), lambda i: (0, 0, 0)),
            pl.BlockSpec((C, C), lambda i: (0, 0)),
            pl.BlockSpec((1, C), lambda i: (0, 0)),
        ],
        out_specs=pl.BlockSpec((BLK, C), lambda i: (i, 0)),
        out_shape=jax.ShapeDtypeStruct((N, C), jnp.float32),
    )(x, agg, agg, agg, agg, weight, rw, bias)


@jax.jit
def kernel(x, edge_index, edge_type, weight, root_w, root_b):
    src = edge_index[0]
    dst = edge_index[1]

    xr = jnp.concatenate([x[:, :H], x[:, H:]], axis=0)
    pad = E_PAD - E
    gix_r = jnp.concatenate([src, jnp.zeros((pad,), jnp.int32)]).reshape(
        NS, NB, B)
    six = edge_type * N + dst
    trash = T * N + jnp.arange(pad, dtype=jnp.int32) % (ROWS - T * N)
    six_r = jnp.concatenate([six, trash]).reshape(NS, NB, B)

    agg = _sc_segment_sum(xr, gix_r, six_r)
    return _tc_dense(x, agg, weight, root_w.T, root_b.reshape(1, C))
